# Initial kernel scaffold; baseline (speedup 1.0000x reference)
import jax
import jax.numpy as jnp
from jax.experimental import pallas as pl


def kernel(x, edge_index, W_in, b_in, W0, b0, g0, be0, W1, b1, g1, be1, W2, b2, g2, be2, gates):
    raise NotImplementedError("write your pallas kernel here")


# speedup vs baseline: 1.8609x; 1.8609x over previous
import functools

import jax
import jax.numpy as jnp
from jax import lax
from jax.experimental import pallas as pl
from jax.experimental.pallas import tpu as pltpu
from jax.experimental.pallas import tpu_sc as plsc

NC = 2
NS = 16
L = 16
NW = NC * NS

_N = 10000
_E = 320000
_H = 128
R = 320
NPAD = NW * R
CH = 8000
NCH = _E // CH
B = 512
ECAP = 330240

_mesh = plsc.VectorSubcoreMesh(
    core_axis_name="c", subcore_axis_name="s", num_cores=NC, num_subcores=NS
)


@functools.partial(
    pl.kernel,
    out_type=(
        jax.ShapeDtypeStruct((NW * ECAP,), jnp.int32),
        jax.ShapeDtypeStruct((NW * ECAP,), jnp.int32),
        jax.ShapeDtypeStruct((NW * L,), jnp.int32),
    ),
    mesh=_mesh,
    scratch_types=[
        pltpu.VMEM((CH,), jnp.int32),
        pltpu.VMEM((CH,), jnp.int32),
        pltpu.VMEM((CH + 32,), jnp.int32),
        pltpu.VMEM((CH + 32,), jnp.int32),
        pltpu.VMEM((L,), jnp.int32),
    ],
    compiler_params=pltpu.CompilerParams(needs_layout_passes=False),
)
def _partition(dst_hbm, src_hbm, srcl_hbm, dll_hbm, cnt_hbm, dbuf, sbuf, csb, cdb, cstage):
    wid = lax.axis_index("s") * NC + lax.axis_index("c")
    lo = wid * R
    dummy_s = jnp.zeros((L,), jnp.int32)
    dummy_d = jnp.full((L,), R, jnp.int32)
    iota = lax.iota(jnp.int32, L)

    def chunk_body(ci, off):
        pltpu.sync_copy(dst_hbm.at[pl.ds(ci * CH, CH)], dbuf)
        pltpu.sync_copy(src_hbm.at[pl.ds(ci * CH, CH)], sbuf)

        def vbody(i, cnt):
            dv = dbuf[pl.ds(i * L, L)]
            sv = sbuf[pl.ds(i * L, L)]
            m = (dv >= lo) & (dv < lo + R)
            mi = m.astype(jnp.int32)
            incl = plsc.cumsum(mi)
            pos = cnt + incl - mi
            plsc.store_scatter(csb, [pos], sv, mask=m)
            plsc.store_scatter(cdb, [pos], dv - lo, mask=m)
            return cnt + jnp.max(incl)

        cnt = lax.fori_loop(0, CH // L, vbody, 0)
        off = pl.multiple_of(off, L)
        plsc.store_scatter(csb, [cnt + iota], dummy_s)
        plsc.store_scatter(cdb, [cnt + iota], dummy_d)
        pltpu.sync_copy(csb.at[pl.ds(0, CH + L)], srcl_hbm.at[pl.ds(wid * ECAP + off, CH + L)])
        pltpu.sync_copy(cdb.at[pl.ds(0, CH + L)], dll_hbm.at[pl.ds(wid * ECAP + off, CH + L)])
        return off + ((cnt + (L - 1)) // L) * L

    off = pl.multiple_of(lax.fori_loop(0, NCH, chunk_body, 0), L)

    def fill(k, _):
        csb[pl.ds(k * L, L)] = dummy_s
        cdb[pl.ds(k * L, L)] = dummy_d
        return 0

    lax.fori_loop(0, B // L, fill, 0)
    pltpu.sync_copy(csb.at[pl.ds(0, B)], srcl_hbm.at[pl.ds(wid * ECAP + off, B)])
    pltpu.sync_copy(cdb.at[pl.ds(0, B)], dll_hbm.at[pl.ds(wid * ECAP + off, B)])
    cstage[...] = jnp.zeros((L,), jnp.int32) + off
    pltpu.sync_copy(cstage, cnt_hbm.at[pl.ds(wid * L, L)])


@functools.partial(
    pl.kernel,
    out_type=jax.ShapeDtypeStruct((NPAD * _H,), jnp.float32),
    mesh=_mesh,
    scratch_types=[
        pltpu.VMEM(((R + 1) * _H,), jnp.float32),
        pltpu.VMEM((B,), jnp.int32),
        pltpu.VMEM((B, _H), jnp.float32),
        pltpu.VMEM((B,), jnp.int32),
        pltpu.VMEM((L,), jnp.int32),
        pltpu.SemaphoreType.DMA,
    ],
    compiler_params=pltpu.CompilerParams(needs_layout_passes=False),
)
def _segmax(h_hbm, srcl_hbm, dll_hbm, cnt_hbm, agg_hbm, aggl, sidx, rows, dls, cs, sem):
    wid = lax.axis_index("s") * NC + lax.axis_index("c")
    zero = jnp.zeros((L,), jnp.float32)

    def zbody(k, _):
        aggl[pl.ds(k * L, L)] = zero
        return 0

    lax.fori_loop(0, (R + 1) * _H // L, zbody, 0)

    pltpu.sync_copy(cnt_hbm.at[pl.ds(wid * L, L)], cs)
    total = cs[...][0]
    nb = (total + (B - 1)) // B

    def bbody(bi, _):
        base = pl.multiple_of(bi * B, B)
        pltpu.sync_copy(srcl_hbm.at[pl.ds(wid * ECAP + base, B)], sidx)
        pltpu.sync_copy(dll_hbm.at[pl.ds(wid * ECAP + base, B)], dls)
        pltpu.async_copy(h_hbm.at[sidx], rows, sem).wait()
        rem = jnp.minimum(B, total - base)

        def gbody(g, _):
            dv16 = dls[pl.ds(g * L, L)]
            for k in range(L):
                j = g * L + k
                bofs = pl.multiple_of(dv16[k] * _H, L)
                for f in range(_H // L):
                    cur = aggl[pl.ds(bofs + f * L, L)]
                    mrow = rows[j, pl.ds(f * L, L)]
                    aggl[pl.ds(bofs + f * L, L)] = jnp.maximum(cur, mrow)
            return 0

        lax.fori_loop(0, rem // L, gbody, 0)
        return 0

    lax.fori_loop(0, nb, bbody, 0)
    pltpu.sync_copy(aggl.at[pl.ds(0, R * _H)], agg_hbm.at[pl.ds(wid * R * _H, R * _H)])


def _h0_body(x_ref, w_ref, b_ref, o_ref):
    h = jnp.dot(x_ref[...], w_ref[...], preferred_element_type=jnp.float32)
    o_ref[...] = jnp.maximum(h + b_ref[...], 0.0)


_MB = 1024


def _h0(xp, W_in, b_in):
    return pl.pallas_call(
        _h0_body,
        grid=(NPAD // _MB,),
        in_specs=[
            pl.BlockSpec((_MB, _H), lambda i: (i, 0)),
            pl.BlockSpec((_H, _H), lambda i: (0, 0)),
            pl.BlockSpec((1, _H), lambda i: (0, 0)),
        ],
        out_specs=pl.BlockSpec((_MB, _H), lambda i: (i, 0)),
        out_shape=jax.ShapeDtypeStruct((NPAD, _H), jnp.float32),
    )(xp, W_in, b_in.reshape(1, _H))


def _hop_body(agg_ref, h_ref, f_ref, w_ref, b_ref, g_ref, be_ref, gate_ref, ho_ref, fo_ref):
    out = jnp.dot(agg_ref[...], w_ref[...], preferred_element_type=jnp.float32) + b_ref[...]
    mu = jnp.mean(out, axis=-1, keepdims=True)
    var = jnp.mean(jnp.square(out - mu), axis=-1, keepdims=True)
    ln = (out - mu) / jnp.sqrt(var + 1e-5) * g_ref[...] + be_ref[...]
    hn = jnp.maximum(ln + h_ref[...], 0.0)
    ho_ref[...] = hn
    fo_ref[...] = f_ref[...] + hn * jax.nn.sigmoid(gate_ref[...])


def _hop(agg, h, fused, W, b, g, be, gate):
    vec = pl.BlockSpec((1, _H), lambda i: (0, 0))
    return pl.pallas_call(
        _hop_body,
        grid=(NPAD // _MB,),
        in_specs=[
            pl.BlockSpec((_MB, _H), lambda i: (i, 0)),
            pl.BlockSpec((_MB, _H), lambda i: (i, 0)),
            pl.BlockSpec((_MB, _H), lambda i: (i, 0)),
            pl.BlockSpec((_H, _H), lambda i: (0, 0)),
            vec, vec, vec, vec,
        ],
        out_specs=[
            pl.BlockSpec((_MB, _H), lambda i: (i, 0)),
            pl.BlockSpec((_MB, _H), lambda i: (i, 0)),
        ],
        out_shape=[
            jax.ShapeDtypeStruct((NPAD, _H), jnp.float32),
            jax.ShapeDtypeStruct((NPAD, _H), jnp.float32),
        ],
    )(agg, h, fused, W, b.reshape(1, _H), g.reshape(1, _H), be.reshape(1, _H), gate)


def kernel(x, edge_index, W_in, b_in, W0, b0, g0, be0, W1, b1, g1, be1, W2, b2, g2, be2, gates):
    src = edge_index[0]
    dst = edge_index[1]
    srcl, dll, cnts = _partition(dst, src)
    xp = jnp.pad(x, ((0, NPAD - _N), (0, 0)))
    h = _h0(xp, W_in, b_in)
    fused = jnp.zeros((NPAD, _H), jnp.float32)
    hop_params = ((W0, b0, g0, be0), (W1, b1, g1, be1), (W2, b2, g2, be2))
    for i, (W, b, g, be) in enumerate(hop_params):
        agg = _segmax(h, srcl, dll, cnts).reshape(NPAD, _H)
        h, fused = _hop(agg, h, fused, W, b, g, be, gates[i].reshape(1, _H))
    return fused[:_N]


# speedup vs baseline: 2.2277x; 1.1971x over previous
import functools

import jax
import jax.numpy as jnp
from jax import lax
from jax.experimental import pallas as pl
from jax.experimental.pallas import tpu as pltpu
from jax.experimental.pallas import tpu_sc as plsc

NC = 2
NS = 16
L = 16
NW = NC * NS

_N = 10000
_E = 320000
_H = 128
R = 320
NPAD = NW * R
CH = 8000
NCH = _E // CH
B = 256
ECAP = 330240

_mesh = plsc.VectorSubcoreMesh(
    core_axis_name="c", subcore_axis_name="s", num_cores=NC, num_subcores=NS
)


@functools.partial(
    pl.kernel,
    out_type=(
        jax.ShapeDtypeStruct((NW * ECAP,), jnp.int32),
        jax.ShapeDtypeStruct((NW * L,), jnp.int32),
    ),
    mesh=_mesh,
    scratch_types=[
        pltpu.VMEM((CH,), jnp.int32),
        pltpu.VMEM((CH,), jnp.int32),
        pltpu.VMEM((CH + 32,), jnp.int32),
        pltpu.VMEM((L,), jnp.int32),
    ],
    compiler_params=pltpu.CompilerParams(needs_layout_passes=False),
)
def _partition(dst_hbm, src_hbm, pl_hbm, cnt_hbm, dbuf, sbuf, cpb, cstage):
    wid = lax.axis_index("s") * NC + lax.axis_index("c")
    lo = wid * R
    dummy = jnp.full((L,), R << 14, jnp.int32)
    iota = lax.iota(jnp.int32, L)

    def chunk_body(ci, off):
        pltpu.sync_copy(dst_hbm.at[pl.ds(ci * CH, CH)], dbuf)
        pltpu.sync_copy(src_hbm.at[pl.ds(ci * CH, CH)], sbuf)

        def vbody(i, cnt):
            for u in range(2):
                dv = dbuf[pl.ds((2 * i + u) * L, L)]
                sv = sbuf[pl.ds((2 * i + u) * L, L)]
                dl = dv - lo
                m = (dl >= 0) & (dl < R)
                mi = m.astype(jnp.int32)
                incl = plsc.cumsum(mi)
                pos = cnt + incl - mi
                plsc.store_scatter(cpb, [pos], (dl << 14) | sv, mask=m)
                cnt = cnt + plsc.all_reduce_population_count(m)[0]
            return cnt

        cnt = lax.fori_loop(0, CH // (2 * L), vbody, 0)
        off = pl.multiple_of(off, L)
        plsc.store_scatter(cpb, [cnt + iota], dummy)
        pltpu.sync_copy(cpb.at[pl.ds(0, CH + L)], pl_hbm.at[pl.ds(wid * ECAP + off, CH + L)])
        return off + ((cnt + (L - 1)) // L) * L

    off = pl.multiple_of(lax.fori_loop(0, NCH, chunk_body, 0), L)

    def fill(k, _):
        cpb[pl.ds(k * L, L)] = dummy
        return 0

    lax.fori_loop(0, B // L, fill, 0)
    pltpu.sync_copy(cpb.at[pl.ds(0, B)], pl_hbm.at[pl.ds(wid * ECAP + off, B)])
    cstage[...] = jnp.zeros((L,), jnp.int32) + off
    pltpu.sync_copy(cstage, cnt_hbm.at[pl.ds(wid * L, L)])


@functools.partial(
    pl.kernel,
    out_type=jax.ShapeDtypeStruct((NPAD * _H,), jnp.float32),
    mesh=_mesh,
    scratch_types=[
        pltpu.VMEM(((R + 1) * _H,), jnp.float32),
        pltpu.VMEM(((R + 1) * _H,), jnp.float32),
        pltpu.VMEM((B,), jnp.int32),
        pltpu.VMEM((B,), jnp.int32),
        pltpu.VMEM((B,), jnp.int32),
        pltpu.VMEM((B, _H), jnp.float32),
        pltpu.VMEM((L,), jnp.int32),
        pltpu.SemaphoreType.DMA,
    ],
    compiler_params=pltpu.CompilerParams(needs_layout_passes=False),
)
def _segmax(h_hbm, pl_hbm, cnt_hbm, agg_hbm, accA, accB, pb, sidx, dlo, rows, cs, sem):
    wid = lax.axis_index("s") * NC + lax.axis_index("c")
    zero = jnp.zeros((L,), jnp.float32)

    def zbody(k, _):
        accA[pl.ds(k * L, L)] = zero
        accB[pl.ds(k * L, L)] = zero
        return 0

    lax.fori_loop(0, (R + 1) * _H // L, zbody, 0)

    pltpu.sync_copy(cnt_hbm.at[pl.ds(wid * L, L)], cs)
    total = cs[...][0]
    nb = (total + (B - 1)) // B

    def bbody(bi, _):
        base = pl.multiple_of(bi * B, B)
        pltpu.sync_copy(pl_hbm.at[pl.ds(wid * ECAP + base, B)], pb)

        def ubody(k, _):
            v = pb[pl.ds(k * L, L)]
            sidx[pl.ds(k * L, L)] = v & ((1 << 14) - 1)
            dlo[pl.ds(k * L, L)] = (v >> 14) << 7
            return 0

        lax.fori_loop(0, B // L, ubody, 0)
        pltpu.async_copy(h_hbm.at[sidx], rows, sem).wait()
        rem = jnp.minimum(B, total - base)

        def gbody(g, _):
            dv16 = dlo[pl.ds(g * L, L)]
            for k in range(L):
                j = g * L + k
                acc = accA if k % 2 == 0 else accB
                bofs = pl.multiple_of(dv16[k], L)
                mv = [rows[j, pl.ds(f * L, L)] for f in range(_H // L)]
                av = [acc[pl.ds(bofs + f * L, L)] for f in range(_H // L)]
                for f in range(_H // L):
                    acc[pl.ds(bofs + f * L, L)] = jnp.maximum(av[f], mv[f])
            return 0

        lax.fori_loop(0, rem // L, gbody, 0)
        return 0

    lax.fori_loop(0, nb, bbody, 0)

    def mbody(k, _):
        accA[pl.ds(k * L, L)] = jnp.maximum(accA[pl.ds(k * L, L)], accB[pl.ds(k * L, L)])
        return 0

    lax.fori_loop(0, R * _H // L, mbody, 0)
    pltpu.sync_copy(accA.at[pl.ds(0, R * _H)], agg_hbm.at[pl.ds(wid * R * _H, R * _H)])


def _h0_body(x_ref, w_ref, b_ref, o_ref):
    h = jnp.dot(x_ref[...], w_ref[...], preferred_element_type=jnp.float32)
    o_ref[...] = jnp.maximum(h + b_ref[...], 0.0)


_MB = 1024


def _h0(xp, W_in, b_in):
    return pl.pallas_call(
        _h0_body,
        grid=(NPAD // _MB,),
        in_specs=[
            pl.BlockSpec((_MB, _H), lambda i: (i, 0)),
            pl.BlockSpec((_H, _H), lambda i: (0, 0)),
            pl.BlockSpec((1, _H), lambda i: (0, 0)),
        ],
        out_specs=pl.BlockSpec((_MB, _H), lambda i: (i, 0)),
        out_shape=jax.ShapeDtypeStruct((NPAD, _H), jnp.float32),
    )(xp, W_in, b_in.reshape(1, _H))


def _hop_body(agg_ref, h_ref, f_ref, w_ref, b_ref, g_ref, be_ref, gate_ref, ho_ref, fo_ref):
    out = jnp.dot(agg_ref[...], w_ref[...], preferred_element_type=jnp.float32) + b_ref[...]
    mu = jnp.mean(out, axis=-1, keepdims=True)
    var = jnp.mean(jnp.square(out - mu), axis=-1, keepdims=True)
    ln = (out - mu) / jnp.sqrt(var + 1e-5) * g_ref[...] + be_ref[...]
    hn = jnp.maximum(ln + h_ref[...], 0.0)
    ho_ref[...] = hn
    fo_ref[...] = f_ref[...] + hn * jax.nn.sigmoid(gate_ref[...])


def _hop(agg, h, fused, W, b, g, be, gate):
    vec = pl.BlockSpec((1, _H), lambda i: (0, 0))
    return pl.pallas_call(
        _hop_body,
        grid=(NPAD // _MB,),
        in_specs=[
            pl.BlockSpec((_MB, _H), lambda i: (i, 0)),
            pl.BlockSpec((_MB, _H), lambda i: (i, 0)),
            pl.BlockSpec((_MB, _H), lambda i: (i, 0)),
            pl.BlockSpec((_H, _H), lambda i: (0, 0)),
            vec, vec, vec, vec,
        ],
        out_specs=[
            pl.BlockSpec((_MB, _H), lambda i: (i, 0)),
            pl.BlockSpec((_MB, _H), lambda i: (i, 0)),
        ],
        out_shape=[
            jax.ShapeDtypeStruct((NPAD, _H), jnp.float32),
            jax.ShapeDtypeStruct((NPAD, _H), jnp.float32),
        ],
    )(agg, h, fused, W, b.reshape(1, _H), g.reshape(1, _H), be.reshape(1, _H), gate)


def kernel(x, edge_index, W_in, b_in, W0, b0, g0, be0, W1, b1, g1, be1, W2, b2, g2, be2, gates):
    src = edge_index[0]
    dst = edge_index[1]
    plist, cnts = _partition(dst, src)
    xp = jnp.pad(x, ((0, NPAD - _N), (0, 0)))
    h = _h0(xp, W_in, b_in)
    fused = jnp.zeros((NPAD, _H), jnp.float32)
    hop_params = ((W0, b0, g0, be0), (W1, b1, g1, be1), (W2, b2, g2, be2))
    for i, (W, b, g, be) in enumerate(hop_params):
        agg = _segmax(h, plist, cnts).reshape(NPAD, _H)
        h, fused = _hop(agg, h, fused, W, b, g, be, gates[i].reshape(1, _H))
    return fused[:_N]


# speedup vs baseline: 3.3832x; 1.5187x over previous
import functools

import jax
import jax.numpy as jnp
from jax import lax
from jax.experimental import pallas as pl
from jax.experimental.pallas import tpu as pltpu
from jax.experimental.pallas import tpu_sc as plsc

NC = 2
NS = 16
L = 16
NW = NC * NS

_N = 10000
_E = 320000
_H = 128
_HP = _H // 2
R = 320
NPAD = NW * R
CH = 8000
NCH = _E // CH
B = 256
ECAP = 330240

_mesh = plsc.VectorSubcoreMesh(
    core_axis_name="c", subcore_axis_name="s", num_cores=NC, num_subcores=NS
)


@functools.partial(
    pl.kernel,
    out_type=(
        jax.ShapeDtypeStruct((NW * ECAP,), jnp.int32),
        jax.ShapeDtypeStruct((NW * L,), jnp.int32),
    ),
    mesh=_mesh,
    scratch_types=[
        pltpu.VMEM((CH,), jnp.int32),
        pltpu.VMEM((CH,), jnp.int32),
        pltpu.VMEM((CH + 32,), jnp.int32),
        pltpu.VMEM((L,), jnp.int32),
    ],
    compiler_params=pltpu.CompilerParams(needs_layout_passes=False),
)
def _partition(dst_hbm, src_hbm, pl_hbm, cnt_hbm, dbuf, sbuf, cpb, cstage):
    wid = lax.axis_index("s") * NC + lax.axis_index("c")
    lo = wid * R
    dummy = jnp.full((L,), R << 14, jnp.int32)
    iota = lax.iota(jnp.int32, L)

    def chunk_body(ci, off):
        pltpu.sync_copy(dst_hbm.at[pl.ds(ci * CH, CH)], dbuf)
        pltpu.sync_copy(src_hbm.at[pl.ds(ci * CH, CH)], sbuf)

        def vbody(i, cnt):
            for u in range(2):
                dv = dbuf[pl.ds((2 * i + u) * L, L)]
                sv = sbuf[pl.ds((2 * i + u) * L, L)]
                dl = dv - lo
                m = (dl >= 0) & (dl < R)
                mi = m.astype(jnp.int32)
                incl = plsc.cumsum(mi)
                pos = cnt + incl - mi
                plsc.store_scatter(cpb, [pos], (dl << 14) | sv, mask=m)
                cnt = cnt + plsc.all_reduce_population_count(m)[0]
            return cnt

        cnt = lax.fori_loop(0, CH // (2 * L), vbody, 0)
        off = pl.multiple_of(off, L)
        plsc.store_scatter(cpb, [cnt + iota], dummy)
        pltpu.sync_copy(cpb.at[pl.ds(0, CH + L)], pl_hbm.at[pl.ds(wid * ECAP + off, CH + L)])
        return off + ((cnt + (L - 1)) // L) * L

    off = pl.multiple_of(lax.fori_loop(0, NCH, chunk_body, 0), L)

    def fill(k, _):
        cpb[pl.ds(k * L, L)] = dummy
        return 0

    lax.fori_loop(0, B // L, fill, 0)
    pltpu.sync_copy(cpb.at[pl.ds(0, B)], pl_hbm.at[pl.ds(wid * ECAP + off, B)])
    cstage[...] = jnp.zeros((L,), jnp.int32) + off
    pltpu.sync_copy(cstage, cnt_hbm.at[pl.ds(wid * L, L)])


@functools.partial(
    pl.kernel,
    out_type=jax.ShapeDtypeStruct((NPAD * _HP,), jnp.int32),
    mesh=_mesh,
    scratch_types=[
        pltpu.VMEM(((R + 1) * _HP,), jnp.int32),
        pltpu.VMEM(((R + 1) * _HP,), jnp.int32),
        pltpu.VMEM((B,), jnp.int32),
        pltpu.VMEM((B,), jnp.int32),
        pltpu.VMEM((B,), jnp.int32),
        pltpu.VMEM((B, _HP), jnp.int32),
        pltpu.VMEM((L,), jnp.int32),
        pltpu.SemaphoreType.DMA,
    ],
    compiler_params=pltpu.CompilerParams(
        needs_layout_passes=False, use_tc_tiling_on_sc=False
    ),
)
def _segmax(h_hbm, pl_hbm, cnt_hbm, agg_hbm, accA, accB, pb, sidx, dlo, rows, cs, sem):
    wid = lax.axis_index("s") * NC + lax.axis_index("c")
    zero = jnp.zeros((L,), jnp.int32)

    def zbody(k, _):
        for u in range(4):
            accA[pl.ds((4 * k + u) * L, L)] = zero
            accB[pl.ds((4 * k + u) * L, L)] = zero
        return 0

    lax.fori_loop(0, (R + 1) * _HP // (4 * L), zbody, 0)

    pltpu.sync_copy(cnt_hbm.at[pl.ds(wid * L, L)], cs)
    total = cs[...][0]
    nb = (total + (B - 1)) // B

    def bbody(bi, _):
        base = pl.multiple_of(bi * B, B)
        pltpu.sync_copy(pl_hbm.at[pl.ds(wid * ECAP + base, B)], pb)

        def ubody(k, _):
            v = pb[pl.ds(k * L, L)]
            sidx[pl.ds(k * L, L)] = v & ((1 << 14) - 1)
            dlo[pl.ds(k * L, L)] = (v >> 14) << 6
            return 0

        lax.fori_loop(0, B // L, ubody, 0)
        pltpu.async_copy(h_hbm.at[sidx], rows, sem).wait()
        rem = jnp.minimum(B, total - base)

        def gbody(g, _):
            dv16 = dlo[pl.ds(g * L, L)]
            for k in range(L):
                j = g * L + k
                acc = accA if k % 2 == 0 else accB
                bofs = pl.multiple_of(dv16[k], L)
                mv = [rows[j, pl.ds(f * L, L)] for f in range(_HP // L)]
                av = [acc[pl.ds(bofs + f * L, L)] for f in range(_HP // L)]
                for f in range(_HP // L):
                    mx = jnp.maximum(
                        plsc.bitcast(av[f], jnp.bfloat16),
                        plsc.bitcast(mv[f], jnp.bfloat16),
                    )
                    acc[pl.ds(bofs + f * L, L)] = plsc.bitcast(mx, jnp.int32)
            return 0

        lax.fori_loop(0, rem // L, gbody, 0)
        return 0

    lax.fori_loop(0, nb, bbody, 0)

    def mbody(k, _):
        for u in range(4):
            d = pl.ds((4 * k + u) * L, L)
            mx = jnp.maximum(
                plsc.bitcast(accA[d], jnp.bfloat16),
                plsc.bitcast(accB[d], jnp.bfloat16),
            )
            accA[d] = plsc.bitcast(mx, jnp.int32)
        return 0

    lax.fori_loop(0, R * _HP // (4 * L), mbody, 0)
    pltpu.sync_copy(accA.at[pl.ds(0, R * _HP)], agg_hbm.at[pl.ds(wid * R * _HP, R * _HP)])


def _h0_body(x_ref, w_ref, b_ref, o_ref, op_ref):
    h = jnp.dot(x_ref[...], w_ref[...], preferred_element_type=jnp.float32)
    hn = jnp.maximum(h + b_ref[...], 0.0)
    o_ref[...] = hn
    op_ref[...] = hn.astype(jnp.bfloat16)


_MB = 1024


def _h0(xp, W_in, b_in):
    return pl.pallas_call(
        _h0_body,
        grid=(NPAD // _MB,),
        in_specs=[
            pl.BlockSpec((_MB, _H), lambda i: (i, 0)),
            pl.BlockSpec((_H, _H), lambda i: (0, 0)),
            pl.BlockSpec((1, _H), lambda i: (0, 0)),
        ],
        out_specs=[
            pl.BlockSpec((_MB, _H), lambda i: (i, 0)),
            pl.BlockSpec((_MB, _H), lambda i: (i, 0)),
        ],
        out_shape=[
            jax.ShapeDtypeStruct((NPAD, _H), jnp.float32),
            jax.ShapeDtypeStruct((NPAD, _H), jnp.bfloat16),
        ],
    )(xp, W_in, b_in.reshape(1, _H))


def _hop_body(agg_ref, h_ref, f_ref, w_ref, b_ref, g_ref, be_ref, gate_ref,
              ho_ref, hp_ref, fo_ref):
    agg = agg_ref[...].astype(jnp.float32)
    out = jnp.dot(agg, w_ref[...], preferred_element_type=jnp.float32) + b_ref[...]
    mu = jnp.mean(out, axis=-1, keepdims=True)
    var = jnp.mean(jnp.square(out - mu), axis=-1, keepdims=True)
    ln = (out - mu) / jnp.sqrt(var + 1e-5) * g_ref[...] + be_ref[...]
    hn = jnp.maximum(ln + h_ref[...], 0.0)
    ho_ref[...] = hn
    hp_ref[...] = hn.astype(jnp.bfloat16)
    fo_ref[...] = f_ref[...] + hn * jax.nn.sigmoid(gate_ref[...])


def _hop(agg16, h, fused, W, b, g, be, gate):
    vec = pl.BlockSpec((1, _H), lambda i: (0, 0))
    blk = pl.BlockSpec((_MB, _H), lambda i: (i, 0))
    return pl.pallas_call(
        _hop_body,
        grid=(NPAD // _MB,),
        in_specs=[blk, blk, blk, pl.BlockSpec((_H, _H), lambda i: (0, 0)),
                  vec, vec, vec, vec],
        out_specs=[blk, blk, blk],
        out_shape=[
            jax.ShapeDtypeStruct((NPAD, _H), jnp.float32),
            jax.ShapeDtypeStruct((NPAD, _H), jnp.bfloat16),
            jax.ShapeDtypeStruct((NPAD, _H), jnp.float32),
        ],
    )(agg16, h, fused, W, b.reshape(1, _H), g.reshape(1, _H), be.reshape(1, _H), gate)


def _pack(hb16):
    return jax.lax.bitcast_convert_type(
        hb16.reshape(NPAD, _HP, 2), jnp.int32).reshape(NPAD, _HP)


def _unpack(agg_pk):
    return jax.lax.bitcast_convert_type(
        agg_pk.reshape(NPAD, _HP), jnp.bfloat16).reshape(NPAD, _H)


def kernel(x, edge_index, W_in, b_in, W0, b0, g0, be0, W1, b1, g1, be1, W2, b2, g2, be2, gates):
    src = edge_index[0]
    dst = edge_index[1]
    plist, cnts = _partition(dst, src)
    xp = jnp.pad(x, ((0, NPAD - _N), (0, 0)))
    h, hb = _h0(xp, W_in, b_in)
    fused = jnp.zeros((NPAD, _H), jnp.float32)
    hop_params = ((W0, b0, g0, be0), (W1, b1, g1, be1), (W2, b2, g2, be2))
    for i, (W, b, g, be) in enumerate(hop_params):
        agg_pk = _segmax(_pack(hb), plist, cnts)
        h, hb, fused = _hop(_unpack(agg_pk), h, fused, W, b, g, be, gates[i].reshape(1, _H))
    return fused[:_N]


# speedup vs baseline: 3.4112x; 1.0083x over previous
import functools

import jax
import jax.numpy as jnp
from jax import lax
from jax.experimental import pallas as pl
from jax.experimental.pallas import tpu as pltpu
from jax.experimental.pallas import tpu_sc as plsc

NC = 2
NS = 16
L = 16
NW = NC * NS

_N = 10000
_E = 320000
_H = 128
_HP = _H // 2
R = 320
NPAD = NW * R
CH = 8000
NCH = _E // CH
B = 256
ECAP = 330240

_mesh = plsc.VectorSubcoreMesh(
    core_axis_name="c", subcore_axis_name="s", num_cores=NC, num_subcores=NS
)


@functools.partial(
    pl.kernel,
    out_type=(
        jax.ShapeDtypeStruct((NW * ECAP,), jnp.int32),
        jax.ShapeDtypeStruct((NW * L,), jnp.int32),
    ),
    mesh=_mesh,
    scratch_types=[
        pltpu.VMEM((CH,), jnp.int32),
        pltpu.VMEM((CH,), jnp.int32),
        pltpu.VMEM((CH + 32,), jnp.int32),
        pltpu.VMEM((L,), jnp.int32),
    ],
    compiler_params=pltpu.CompilerParams(needs_layout_passes=False),
)
def _partition(dst_hbm, src_hbm, pl_hbm, cnt_hbm, dbuf, sbuf, cpb, cstage):
    wid = lax.axis_index("s") * NC + lax.axis_index("c")
    lo = wid * R
    dummy = jnp.full((L,), R << 14, jnp.int32)
    iota = lax.iota(jnp.int32, L)

    def chunk_body(ci, off):
        pltpu.sync_copy(dst_hbm.at[pl.ds(ci * CH, CH)], dbuf)
        pltpu.sync_copy(src_hbm.at[pl.ds(ci * CH, CH)], sbuf)

        def vbody(i, cntv):
            for u in range(4):
                dv = dbuf[pl.ds((4 * i + u) * L, L)]
                sv = sbuf[pl.ds((4 * i + u) * L, L)]
                dl = dv - lo
                m = (dl >= 0) & (dl < R)
                mi = m.astype(jnp.int32)
                incl = plsc.cumsum(mi)
                pos = cntv + (incl - mi)
                plsc.store_scatter(cpb, [pos], (dl << 14) | sv, mask=m)
                cntv = cntv + plsc.all_reduce_population_count(m)
            return cntv

        cntv = lax.fori_loop(0, CH // (4 * L), vbody, jnp.zeros((L,), jnp.int32))
        cnt = cntv[0]
        off = pl.multiple_of(off, L)
        plsc.store_scatter(cpb, [cnt + iota], dummy)
        pltpu.sync_copy(cpb.at[pl.ds(0, CH + L)], pl_hbm.at[pl.ds(wid * ECAP + off, CH + L)])
        return off + ((cnt + (L - 1)) // L) * L

    off = pl.multiple_of(lax.fori_loop(0, NCH, chunk_body, 0), L)

    def fill(k, _):
        cpb[pl.ds(k * L, L)] = dummy
        return 0

    lax.fori_loop(0, B // L, fill, 0)
    pltpu.sync_copy(cpb.at[pl.ds(0, B)], pl_hbm.at[pl.ds(wid * ECAP + off, B)])
    cstage[...] = jnp.zeros((L,), jnp.int32) + off
    pltpu.sync_copy(cstage, cnt_hbm.at[pl.ds(wid * L, L)])


@functools.partial(
    pl.kernel,
    out_type=jax.ShapeDtypeStruct((NPAD * _HP,), jnp.int32),
    mesh=_mesh,
    scratch_types=[
        pltpu.VMEM(((R + 1) * _HP,), jnp.int32),
        pltpu.VMEM(((R + 1) * _HP,), jnp.int32),
        pltpu.VMEM((B,), jnp.int32),
        pltpu.VMEM((B,), jnp.int32),
        pltpu.VMEM((B,), jnp.int32),
        pltpu.VMEM((B, _HP), jnp.int32),
        pltpu.VMEM((L,), jnp.int32),
        pltpu.SemaphoreType.DMA,
    ],
    compiler_params=pltpu.CompilerParams(
        needs_layout_passes=False, use_tc_tiling_on_sc=False
    ),
)
def _segmax(h_hbm, pl_hbm, cnt_hbm, agg_hbm, accA, accB, pb, sidx, dlo, rows, cs, sem):
    wid = lax.axis_index("s") * NC + lax.axis_index("c")
    zero = jnp.zeros((L,), jnp.int32)

    def zbody(k, _):
        for u in range(4):
            accA[pl.ds((4 * k + u) * L, L)] = zero
            accB[pl.ds((4 * k + u) * L, L)] = zero
        return 0

    lax.fori_loop(0, (R + 1) * _HP // (4 * L), zbody, 0)

    pltpu.sync_copy(cnt_hbm.at[pl.ds(wid * L, L)], cs)
    total = cs[...][0]
    nb = (total + (B - 1)) // B

    def bbody(bi, _):
        base = pl.multiple_of(bi * B, B)
        pltpu.sync_copy(pl_hbm.at[pl.ds(wid * ECAP + base, B)], pb)

        def ubody(k, _):
            v = pb[pl.ds(k * L, L)]
            sidx[pl.ds(k * L, L)] = v & ((1 << 14) - 1)
            dlo[pl.ds(k * L, L)] = (v >> 14) << 6
            return 0

        lax.fori_loop(0, B // L, ubody, 0)
        pltpu.async_copy(h_hbm.at[sidx], rows, sem).wait()
        rem = jnp.minimum(B, total - base)

        def gbody(g, _):
            dv16 = dlo[pl.ds(g * L, L)]
            for k in range(L):
                j = g * L + k
                acc = accA if k % 2 == 0 else accB
                bofs = pl.multiple_of(dv16[k], L)
                mv = [rows[j, pl.ds(f * L, L)] for f in range(_HP // L)]
                av = [acc[pl.ds(bofs + f * L, L)] for f in range(_HP // L)]
                for f in range(_HP // L):
                    mx = jnp.maximum(
                        plsc.bitcast(av[f], jnp.bfloat16),
                        plsc.bitcast(mv[f], jnp.bfloat16),
                    )
                    acc[pl.ds(bofs + f * L, L)] = plsc.bitcast(mx, jnp.int32)
            return 0

        lax.fori_loop(0, rem // L, gbody, 0)
        return 0

    lax.fori_loop(0, nb, bbody, 0)

    def mbody(k, _):
        for u in range(4):
            d = pl.ds((4 * k + u) * L, L)
            mx = jnp.maximum(
                plsc.bitcast(accA[d], jnp.bfloat16),
                plsc.bitcast(accB[d], jnp.bfloat16),
            )
            accA[d] = plsc.bitcast(mx, jnp.int32)
        return 0

    lax.fori_loop(0, R * _HP // (4 * L), mbody, 0)
    pltpu.sync_copy(accA.at[pl.ds(0, R * _HP)], agg_hbm.at[pl.ds(wid * R * _HP, R * _HP)])


def _h0_body(x_ref, w_ref, b_ref, o_ref, op_ref):
    h = jnp.dot(x_ref[...], w_ref[...], preferred_element_type=jnp.float32)
    hn = jnp.maximum(h + b_ref[...], 0.0)
    o_ref[...] = hn
    op_ref[...] = hn.astype(jnp.bfloat16)


_MB = 1024


def _h0(xp, W_in, b_in):
    return pl.pallas_call(
        _h0_body,
        grid=(NPAD // _MB,),
        in_specs=[
            pl.BlockSpec((_MB, _H), lambda i: (i, 0)),
            pl.BlockSpec((_H, _H), lambda i: (0, 0)),
            pl.BlockSpec((1, _H), lambda i: (0, 0)),
        ],
        out_specs=[
            pl.BlockSpec((_MB, _H), lambda i: (i, 0)),
            pl.BlockSpec((_MB, _H), lambda i: (i, 0)),
        ],
        out_shape=[
            jax.ShapeDtypeStruct((NPAD, _H), jnp.float32),
            jax.ShapeDtypeStruct((NPAD, _H), jnp.bfloat16),
        ],
    )(xp, W_in, b_in.reshape(1, _H))


def _hop_body(agg_ref, h_ref, f_ref, w_ref, b_ref, g_ref, be_ref, gate_ref,
              ho_ref, hp_ref, fo_ref):
    agg = agg_ref[...].astype(jnp.float32)
    out = jnp.dot(agg, w_ref[...], preferred_element_type=jnp.float32) + b_ref[...]
    mu = jnp.mean(out, axis=-1, keepdims=True)
    var = jnp.mean(jnp.square(out - mu), axis=-1, keepdims=True)
    ln = (out - mu) / jnp.sqrt(var + 1e-5) * g_ref[...] + be_ref[...]
    hn = jnp.maximum(ln + h_ref[...], 0.0)
    ho_ref[...] = hn
    hp_ref[...] = hn.astype(jnp.bfloat16)
    fo_ref[...] = f_ref[...] + hn * jax.nn.sigmoid(gate_ref[...])


def _hop(agg16, h, fused, W, b, g, be, gate):
    vec = pl.BlockSpec((1, _H), lambda i: (0, 0))
    blk = pl.BlockSpec((_MB, _H), lambda i: (i, 0))
    return pl.pallas_call(
        _hop_body,
        grid=(NPAD // _MB,),
        in_specs=[blk, blk, blk, pl.BlockSpec((_H, _H), lambda i: (0, 0)),
                  vec, vec, vec, vec],
        out_specs=[blk, blk, blk],
        out_shape=[
            jax.ShapeDtypeStruct((NPAD, _H), jnp.float32),
            jax.ShapeDtypeStruct((NPAD, _H), jnp.bfloat16),
            jax.ShapeDtypeStruct((NPAD, _H), jnp.float32),
        ],
    )(agg16, h, fused, W, b.reshape(1, _H), g.reshape(1, _H), be.reshape(1, _H), gate)


def _pack(hb16):
    return jax.lax.bitcast_convert_type(
        hb16.reshape(NPAD, _HP, 2), jnp.int32).reshape(NPAD, _HP)


def _unpack(agg_pk):
    return jax.lax.bitcast_convert_type(
        agg_pk.reshape(NPAD, _HP), jnp.bfloat16).reshape(NPAD, _H)


def kernel(x, edge_index, W_in, b_in, W0, b0, g0, be0, W1, b1, g1, be1, W2, b2, g2, be2, gates):
    src = edge_index[0]
    dst = edge_index[1]
    plist, cnts = _partition(dst, src)
    xp = jnp.pad(x, ((0, NPAD - _N), (0, 0)))
    h, hb = _h0(xp, W_in, b_in)
    fused = jnp.zeros((NPAD, _H), jnp.float32)
    hop_params = ((W0, b0, g0, be0), (W1, b1, g1, be1), (W2, b2, g2, be2))
    for i, (W, b, g, be) in enumerate(hop_params):
        agg_pk = _segmax(_pack(hb), plist, cnts)
        h, hb, fused = _hop(_unpack(agg_pk), h, fused, W, b, g, be, gates[i].reshape(1, _H))
    return fused[:_N]


# speedup vs baseline: 3.5751x; 1.0481x over previous
import functools

import jax
import jax.numpy as jnp
from jax import lax
from jax.experimental import pallas as pl
from jax.experimental.pallas import tpu as pltpu
from jax.experimental.pallas import tpu_sc as plsc

NC = 2
NS = 16
L = 16
NW = NC * NS

_N = 10000
_E = 320000
_H = 128
_HP = _H // 2
R = 320
NPAD = NW * R
CH = 8000
NCH = _E // CH
B = 256
ECAP = 330240

_mesh = plsc.VectorSubcoreMesh(
    core_axis_name="c", subcore_axis_name="s", num_cores=NC, num_subcores=NS
)


@functools.partial(
    pl.kernel,
    out_type=(
        jax.ShapeDtypeStruct((NW * ECAP,), jnp.int32),
        jax.ShapeDtypeStruct((NW * L,), jnp.int32),
    ),
    mesh=_mesh,
    scratch_types=[
        pltpu.VMEM((CH,), jnp.int32),
        pltpu.VMEM((CH,), jnp.int32),
        pltpu.VMEM((CH + 32,), jnp.int32),
        pltpu.VMEM((L,), jnp.int32),
    ],
    compiler_params=pltpu.CompilerParams(needs_layout_passes=False),
)
def _partition(dst_hbm, src_hbm, pl_hbm, cnt_hbm, dbuf, sbuf, cpb, cstage):
    wid = lax.axis_index("s") * NC + lax.axis_index("c")
    lo = wid * R
    dummy = jnp.full((L,), R << 14, jnp.int32)
    iota = lax.iota(jnp.int32, L)

    def chunk_body(ci, off):
        pltpu.sync_copy(dst_hbm.at[pl.ds(ci * CH, CH)], dbuf)
        pltpu.sync_copy(src_hbm.at[pl.ds(ci * CH, CH)], sbuf)

        def vbody(i, cntv):
            for u in range(4):
                dv = dbuf[pl.ds((4 * i + u) * L, L)]
                sv = sbuf[pl.ds((4 * i + u) * L, L)]
                dl = dv - lo
                m = (dl >= 0) & (dl < R)
                mi = m.astype(jnp.int32)
                incl = plsc.cumsum(mi)
                pos = cntv + (incl - mi)
                plsc.store_scatter(cpb, [pos], (dl << 14) | sv, mask=m)
                cntv = cntv + plsc.all_reduce_population_count(m)
            return cntv

        cntv = lax.fori_loop(0, CH // (4 * L), vbody, jnp.zeros((L,), jnp.int32))
        cnt = cntv[0]
        off = pl.multiple_of(off, L)
        plsc.store_scatter(cpb, [cnt + iota], dummy)
        pltpu.sync_copy(cpb.at[pl.ds(0, CH + L)], pl_hbm.at[pl.ds(wid * ECAP + off, CH + L)])
        return off + ((cnt + (L - 1)) // L) * L

    off = pl.multiple_of(lax.fori_loop(0, NCH, chunk_body, 0), L)

    def fill(k, _):
        cpb[pl.ds(k * L, L)] = dummy
        return 0

    lax.fori_loop(0, B // L, fill, 0)
    pltpu.sync_copy(cpb.at[pl.ds(0, B)], pl_hbm.at[pl.ds(wid * ECAP + off, B)])
    cstage[...] = jnp.zeros((L,), jnp.int32) + off
    pltpu.sync_copy(cstage, cnt_hbm.at[pl.ds(wid * L, L)])


@functools.partial(
    pl.kernel,
    out_type=jax.ShapeDtypeStruct((NPAD * _H,), jnp.bfloat16),
    mesh=_mesh,
    scratch_types=[
        pltpu.VMEM(((R + 1) * _H,), jnp.bfloat16),
        pltpu.VMEM(((R + 1) * _H,), jnp.bfloat16),
        pltpu.VMEM((B,), jnp.int32),
        pltpu.VMEM((B,), jnp.int32),
        pltpu.VMEM((B,), jnp.int32),
        pltpu.VMEM((B, _HP), jnp.int32),
        pltpu.VMEM((L,), jnp.int32),
        pltpu.SemaphoreType.DMA,
    ],
    compiler_params=pltpu.CompilerParams(
        needs_layout_passes=False, use_tc_tiling_on_sc=False
    ),
)
def _segmax(h_hbm, pl_hbm, cnt_hbm, agg_hbm, accA, accB, pb, sidx, dlo, rows, cs, sem):
    wid = lax.axis_index("s") * NC + lax.axis_index("c")
    zero = jnp.zeros((2 * L,), jnp.bfloat16)

    def zbody(k, _):
        for u in range(4):
            accA[pl.ds((4 * k + u) * 2 * L, 2 * L)] = zero
            accB[pl.ds((4 * k + u) * 2 * L, 2 * L)] = zero
        return 0

    lax.fori_loop(0, (R + 1) * _H // (8 * L), zbody, 0)

    pltpu.sync_copy(cnt_hbm.at[pl.ds(wid * L, L)], cs)
    total = cs[...][0]
    nb = (total + (B - 1)) // B

    def bbody(bi, _):
        base = pl.multiple_of(bi * B, B)
        pltpu.sync_copy(pl_hbm.at[pl.ds(wid * ECAP + base, B)], pb)

        def ubody(k, _):
            v = pb[pl.ds(k * L, L)]
            sidx[pl.ds(k * L, L)] = v & ((1 << 14) - 1)
            dlo[pl.ds(k * L, L)] = (v >> 14) << 7
            return 0

        lax.fori_loop(0, B // L, ubody, 0)
        pltpu.async_copy(h_hbm.at[sidx], rows, sem).wait()
        rem = jnp.minimum(B, total - base)

        def gbody(g, _):
            dv16 = dlo[pl.ds(g * L, L)]
            for k in range(L):
                j = g * L + k
                acc = accA if k % 2 == 0 else accB
                bofs = pl.multiple_of(dv16[k], 2 * L)
                mv = [plsc.bitcast(rows[j, pl.ds(f * L, L)], jnp.bfloat16)
                      for f in range(_HP // L)]
                av = [acc[pl.ds(bofs + f * 2 * L, 2 * L)] for f in range(_HP // L)]
                for f in range(_HP // L):
                    acc[pl.ds(bofs + f * 2 * L, 2 * L)] = jnp.maximum(av[f], mv[f])
            return 0

        lax.fori_loop(0, rem // L, gbody, 0)
        return 0

    lax.fori_loop(0, nb, bbody, 0)

    def mbody(k, _):
        for u in range(4):
            d = pl.ds((4 * k + u) * 2 * L, 2 * L)
            accA[d] = jnp.maximum(accA[d], accB[d])
        return 0

    lax.fori_loop(0, R * _H // (8 * L), mbody, 0)
    pltpu.sync_copy(accA.at[pl.ds(0, R * _H)],
                    agg_hbm.at[pl.ds(wid * R * _H, R * _H)])


def _h0_body(x_ref, w_ref, b_ref, o_ref, op_ref):
    h = jnp.dot(x_ref[...], w_ref[...], preferred_element_type=jnp.float32)
    hn = jnp.maximum(h + b_ref[...], 0.0)
    o_ref[...] = hn
    op_ref[...] = hn.astype(jnp.bfloat16)


_MB = 1024


def _h0(xp, W_in, b_in):
    return pl.pallas_call(
        _h0_body,
        grid=(NPAD // _MB,),
        in_specs=[
            pl.BlockSpec((_MB, _H), lambda i: (i, 0)),
            pl.BlockSpec((_H, _H), lambda i: (0, 0)),
            pl.BlockSpec((1, _H), lambda i: (0, 0)),
        ],
        out_specs=[
            pl.BlockSpec((_MB, _H), lambda i: (i, 0)),
            pl.BlockSpec((_MB, _H), lambda i: (i, 0)),
        ],
        out_shape=[
            jax.ShapeDtypeStruct((NPAD, _H), jnp.float32),
            jax.ShapeDtypeStruct((NPAD, _H), jnp.bfloat16),
        ],
    )(xp, W_in, b_in.reshape(1, _H))


def _hop_body(agg_ref, h_ref, f_ref, w_ref, b_ref, g_ref, be_ref, gate_ref,
              ho_ref, hp_ref, fo_ref):
    agg = agg_ref[...].astype(jnp.float32)
    out = jnp.dot(agg, w_ref[...], preferred_element_type=jnp.float32) + b_ref[...]
    mu = jnp.mean(out, axis=-1, keepdims=True)
    var = jnp.mean(jnp.square(out - mu), axis=-1, keepdims=True)
    ln = (out - mu) / jnp.sqrt(var + 1e-5) * g_ref[...] + be_ref[...]
    hn = jnp.maximum(ln + h_ref[...], 0.0)
    ho_ref[...] = hn
    hp_ref[...] = hn.astype(jnp.bfloat16)
    fo_ref[...] = f_ref[...] + hn * jax.nn.sigmoid(gate_ref[...])


def _hop(agg16, h, fused, W, b, g, be, gate):
    vec = pl.BlockSpec((1, _H), lambda i: (0, 0))
    blk = pl.BlockSpec((_MB, _H), lambda i: (i, 0))
    return pl.pallas_call(
        _hop_body,
        grid=(NPAD // _MB,),
        in_specs=[blk, blk, blk, pl.BlockSpec((_H, _H), lambda i: (0, 0)),
                  vec, vec, vec, vec],
        out_specs=[blk, blk, blk],
        out_shape=[
            jax.ShapeDtypeStruct((NPAD, _H), jnp.float32),
            jax.ShapeDtypeStruct((NPAD, _H), jnp.bfloat16),
            jax.ShapeDtypeStruct((NPAD, _H), jnp.float32),
        ],
    )(agg16, h, fused, W, b.reshape(1, _H), g.reshape(1, _H), be.reshape(1, _H), gate)


def _pack(hb16):
    return jax.lax.bitcast_convert_type(
        hb16.reshape(NPAD, _HP, 2), jnp.int32).reshape(NPAD, _HP)


def kernel(x, edge_index, W_in, b_in, W0, b0, g0, be0, W1, b1, g1, be1, W2, b2, g2, be2, gates):
    src = edge_index[0]
    dst = edge_index[1]
    plist, cnts = _partition(dst, src)
    xp = jnp.pad(x, ((0, NPAD - _N), (0, 0)))
    h, hb = _h0(xp, W_in, b_in)
    fused = jnp.zeros((NPAD, _H), jnp.float32)
    hop_params = ((W0, b0, g0, be0), (W1, b1, g1, be1), (W2, b2, g2, be2))
    for i, (W, b, g, be) in enumerate(hop_params):
        agg16 = _segmax(_pack(hb), plist, cnts).reshape(NPAD, _H)
        h, hb, fused = _hop(agg16, h, fused, W, b, g, be, gates[i].reshape(1, _H))
    return fused[:_N]
